# Initial kernel scaffold; baseline (speedup 1.0000x reference)
import jax
import jax.numpy as jnp
from jax.experimental import pallas as pl


def kernel(x, edge_index, edge_attr, W, b):
    raise NotImplementedError("write your pallas kernel here")


# speedup vs baseline: 1.0550x; 1.0550x over previous
import functools

import jax
import jax.numpy as jnp
from jax import lax
from jax.experimental import pallas as pl
from jax.experimental.pallas import tpu as pltpu
from jax.experimental.pallas import tpu_sc as plsc

N = 10000
E = 320000
D = 128
OUT = 128

NC = 2
NS = 16
L = 16
NW = NC * NS
EPW = E // NW
C = 80
NCHUNK = EPW // C
GPC = C // L

ROWS_A = 624
ROWS_B = N - 15 * ROWS_A

_mesh = plsc.VectorSubcoreMesh(
    core_axis_name="c", subcore_axis_name="s", num_cores=NC, num_subcores=NS
)


def _iota16():
    return lax.broadcasted_iota(jnp.int32, (L,), 0)


@functools.partial(
    pl.kernel,
    out_type=(
        jax.ShapeDtypeStruct((E,), jnp.float32),
        jax.ShapeDtypeStruct((NC * N,), jnp.float32),
    ),
    mesh=_mesh,
    compiler_params=pltpu.CompilerParams(needs_layout_passes=False),
    scratch_types=[
        pltpu.VMEM((C,), jnp.int32),
        pltpu.VMEM((C,), jnp.int32),
        pltpu.VMEM((C, D), jnp.float32),
        pltpu.VMEM((C, D), jnp.float32),
        pltpu.VMEM((C,), jnp.float32),
        pltpu.VMEM((ROWS_B,), jnp.float32),
        pltpu.VMEM_SHARED((N,), jnp.float32),
        pltpu.SemaphoreType.DMA,
    ],
)
def _k1(x_hbm, src_hbm, dst_hbm, ea_hbm, pe_hbm, pip_hbm,
        src_v, dst_v, rows_v, ea_v, pe_v, zb_v, pi_sh, sem):
    cid = lax.axis_index("c")
    sid = lax.axis_index("s")
    wid = cid * NS + sid
    zero16 = jnp.zeros((L,), jnp.float32)

    @pl.loop(0, ROWS_B // L)
    def _(i):
        zb_v[pl.ds(i * L, L)] = zero16

    r0 = sid * ROWS_A
    nrows = jnp.where(sid == NS - 1, ROWS_B, ROWS_A)

    @pl.when(sid < NS - 1)
    def _():
        pltpu.sync_copy(zb_v.at[pl.ds(0, ROWS_A)], pi_sh.at[pl.ds(r0, ROWS_A)])

    @pl.when(sid == NS - 1)
    def _():
        pltpu.sync_copy(zb_v, pi_sh.at[pl.ds(r0, ROWS_B)])

    plsc.subcore_barrier()

    @pl.loop(0, NCHUNK)
    def _(k):
        off = wid * EPW + k * C
        pltpu.sync_copy(src_hbm.at[pl.ds(off, C)], src_v)
        pltpu.sync_copy(dst_hbm.at[pl.ds(off, C)], dst_v)
        pltpu.async_copy(x_hbm.at[src_v], rows_v, sem).wait()
        pltpu.sync_copy(ea_hbm.at[pl.ds(off, C)], ea_v)

        @pl.loop(0, GPC)
        def _(g):
            base = g * L
            rowids = base + _iota16()

            def dot_step(j, acc):
                jv = jnp.full((L,), j, jnp.int32)
                xv = plsc.load_gather(rows_v, [rowids, jv])
                ev = plsc.load_gather(ea_v, [rowids, jv])
                return acc + xv * ev

            dots = pl.loop(0, D, init_carry=jnp.zeros((L,), jnp.float32),
                           unroll=8)(dot_step)
            pe_v[pl.ds(base, L)] = jnp.exp(dots)

        pltpu.sync_copy(pe_v, pe_hbm.at[pl.ds(off, C)])
        pltpu.sync_copy(pe_v, pi_sh.at[dst_v], add=True)

    plsc.subcore_barrier()

    @pl.when(sid < NS - 1)
    def _():
        pltpu.sync_copy(pi_sh.at[pl.ds(r0, ROWS_A)], zb_v.at[pl.ds(0, ROWS_A)])
        pltpu.sync_copy(zb_v.at[pl.ds(0, ROWS_A)],
                        pip_hbm.at[pl.ds(cid * N + r0, ROWS_A)])

    @pl.when(sid == NS - 1)
    def _():
        pltpu.sync_copy(pi_sh.at[pl.ds(r0, ROWS_B)], zb_v)
        pltpu.sync_copy(zb_v, pip_hbm.at[pl.ds(cid * N + r0, ROWS_B)])

    del nrows


@functools.partial(
    pl.kernel,
    out_type=jax.ShapeDtypeStruct((NC, N, D), jnp.float32),
    mesh=_mesh,
    compiler_params=pltpu.CompilerParams(needs_layout_passes=False),
    scratch_types=[
        pltpu.VMEM((C,), jnp.int32),
        pltpu.VMEM((C,), jnp.int32),
        pltpu.VMEM((C, D), jnp.float32),
        pltpu.VMEM((C,), jnp.float32),
        pltpu.VMEM((N,), jnp.float32),
        pltpu.VMEM((N,), jnp.float32),
        pltpu.VMEM((L, D), jnp.float32),
        pltpu.VMEM_SHARED((N, D), jnp.float32),
        pltpu.SemaphoreType.DMA,
    ],
)
def _k2(x_hbm, src_hbm, dst_hbm, pe_hbm, pip_hbm, hp_hbm,
        src_v, dst_v, rows_v, pe_v, pa_v, pb_v, zb_v, h_sh, sem):
    cid = lax.axis_index("c")
    sid = lax.axis_index("s")
    wid = cid * NS + sid
    iota = _iota16()
    zero16 = jnp.zeros((L,), jnp.float32)

    pltpu.sync_copy(pip_hbm.at[pl.ds(0, N)], pa_v)
    pltpu.sync_copy(pip_hbm.at[pl.ds(N, N)], pb_v)

    @pl.loop(0, D)
    def _(j):
        plsc.store_scatter(zb_v, [iota, jnp.full((L,), j, jnp.int32)], zero16)

    r0 = sid * ROWS_A

    @pl.when(sid < NS - 1)
    def _():
        @pl.loop(0, ROWS_A // L)
        def _(i):
            pltpu.sync_copy(zb_v, h_sh.at[pl.ds(r0 + i * L, L)])

    @pl.when(sid == NS - 1)
    def _():
        @pl.loop(0, ROWS_B // L)
        def _(i):
            pltpu.sync_copy(zb_v, h_sh.at[pl.ds(r0 + i * L, L)])

    plsc.subcore_barrier()

    @pl.loop(0, NCHUNK)
    def _(k):
        off = wid * EPW + k * C
        pltpu.sync_copy(src_hbm.at[pl.ds(off, C)], src_v)
        pltpu.sync_copy(dst_hbm.at[pl.ds(off, C)], dst_v)
        pltpu.async_copy(x_hbm.at[src_v], rows_v, sem).wait()
        pltpu.sync_copy(pe_hbm.at[pl.ds(off, C)], pe_v)

        @pl.loop(0, GPC)
        def _(g):
            base = g * L
            rowids = base + iota
            dsts = plsc.load_gather(dst_v, [iota + base])
            den = (plsc.load_gather(pa_v, [dsts])
                   + plsc.load_gather(pb_v, [dsts]))
            pn = pe_v[pl.ds(base, L)] / den

            @pl.loop(0, D, unroll=8)
            def _(j):
                jv = jnp.full((L,), j, jnp.int32)
                xv = plsc.load_gather(rows_v, [rowids, jv])
                plsc.store_scatter(rows_v, [rowids, jv], xv * pn)

        pltpu.sync_copy(rows_v, h_sh.at[dst_v], add=True)

    plsc.subcore_barrier()

    @pl.when(sid < NS - 1)
    def _():
        @pl.loop(0, ROWS_A // L)
        def _(i):
            rr = r0 + i * L
            pltpu.sync_copy(h_sh.at[pl.ds(rr, L)], zb_v)
            pltpu.sync_copy(zb_v, hp_hbm.at[cid, pl.ds(rr, L)])

    @pl.when(sid == NS - 1)
    def _():
        @pl.loop(0, ROWS_B // L)
        def _(i):
            rr = r0 + i * L
            pltpu.sync_copy(h_sh.at[pl.ds(rr, L)], zb_v)
            pltpu.sync_copy(zb_v, hp_hbm.at[cid, pl.ds(rr, L)])


_RB = 1000


def _mm_body(x_r, h0_r, h1_r, wt_r, b_r, o_r):
    h = h0_r[...] + h1_r[...]
    o_r[...] = (
        jnp.dot(x_r[...], wt_r[0:D, :], preferred_element_type=jnp.float32)
        + jnp.dot(h, wt_r[D:2 * D, :], preferred_element_type=jnp.float32)
        + b_r[...]
    )


_k3 = pl.pallas_call(
    _mm_body,
    grid=(N // _RB,),
    in_specs=[
        pl.BlockSpec((_RB, D), lambda i: (i, 0)),
        pl.BlockSpec((_RB, D), lambda i: (i, 0)),
        pl.BlockSpec((_RB, D), lambda i: (i, 0)),
        pl.BlockSpec((2 * D, OUT), lambda i: (0, 0)),
        pl.BlockSpec((1, OUT), lambda i: (0, 0)),
    ],
    out_specs=pl.BlockSpec((_RB, OUT), lambda i: (i, 0)),
    out_shape=jax.ShapeDtypeStruct((N, OUT), jnp.float32),
)


@jax.jit
def kernel(x, edge_index, edge_attr, W, b):
    src = edge_index[0]
    dst = edge_index[1]
    pe, pip = _k1(x, src, dst, edge_attr)
    hp = _k2(x, src, dst, pe, pip)
    return _k3(x, hp[0], hp[1], W.T, b.reshape(1, OUT))


# speedup vs baseline: 1.2465x; 1.1815x over previous
import functools

import jax
import jax.numpy as jnp
from jax import lax
from jax.experimental import pallas as pl
from jax.experimental.pallas import tpu as pltpu
from jax.experimental.pallas import tpu_sc as plsc

N = 10000
E = 320000
D = 128
OUT = 128

NC = 2
NS = 16
L = 16
NW = NC * NS
EPW = E // NW
C = 80
NCHUNK = EPW // C
GPC = C // L

ROWS_A = 624
ROWS_B = N - 15 * ROWS_A

_mesh = plsc.VectorSubcoreMesh(
    core_axis_name="c", subcore_axis_name="s", num_cores=NC, num_subcores=NS
)
_params = pltpu.CompilerParams(needs_layout_passes=False)


def _iota16():
    return lax.broadcasted_iota(jnp.int32, (L,), 0)


@functools.partial(
    pl.kernel,
    out_type=(
        jax.ShapeDtypeStruct((NC * N,), jnp.float32),
        jax.ShapeDtypeStruct((NC, N, D), jnp.float32),
    ),
    mesh=_mesh,
    compiler_params=_params,
    scratch_types=[
        pltpu.VMEM((2, C), jnp.int32),
        pltpu.VMEM((2, C), jnp.int32),
        pltpu.VMEM((C, D), jnp.float32),
        pltpu.VMEM((C, D), jnp.float32),
        pltpu.VMEM((C, D), jnp.float32),
        pltpu.VMEM((C, D), jnp.float32),
        pltpu.VMEM((2, C), jnp.float32),
        pltpu.VMEM((ROWS_B,), jnp.float32),
        pltpu.VMEM_SHARED((N,), jnp.float32),
        pltpu.VMEM_SHARED((N, D), jnp.float32),
        pltpu.SemaphoreType.DMA,
        pltpu.SemaphoreType.DMA,
        pltpu.SemaphoreType.DMA,
        pltpu.SemaphoreType.DMA,
        pltpu.SemaphoreType.DMA,
        pltpu.SemaphoreType.DMA,
        pltpu.SemaphoreType.DMA,
    ],
)
def _k1(x_hbm, src_hbm, dst_hbm, ea_hbm, pip_hbm, hp_hbm,
        src_v, dst_v, rows0_v, rows1_v, ea0_v, ea1_v, pe_v, zb_v,
        pi_sh, h_sh, semi0, semi1, seme0, seme1, semr0, semr1, semo):
    cid = lax.axis_index("c")
    sid = lax.axis_index("s")
    wid = cid * NS + sid
    iota = _iota16()
    zero16 = jnp.zeros((L,), jnp.float32)
    rows_v = (rows0_v, rows1_v)
    ea_v = (ea0_v, ea1_v)
    semi = (semi0, semi1)
    seme = (seme0, seme1)
    semr = (semr0, semr1)

    r0 = sid * ROWS_A

    def fill(k, b):
        off = wid * EPW + k * C
        pltpu.async_copy(src_hbm.at[pl.ds(off, C)], src_v.at[b], semi[b])
        pltpu.async_copy(dst_hbm.at[pl.ds(off, C)], dst_v.at[b], semi[b])
        pltpu.async_copy(ea_hbm.at[pl.ds(off, C)], ea_v[b], seme[b])

    def wait_idx(k, b):
        off = wid * EPW + k * C
        pltpu.make_async_copy(src_hbm.at[pl.ds(off, C)], src_v.at[b], semi[b]).wait()
        pltpu.make_async_copy(dst_hbm.at[pl.ds(off, C)], dst_v.at[b], semi[b]).wait()

    def start_gather(b):
        pltpu.async_copy(x_hbm.at[src_v.at[b]], rows_v[b], semr[b])

    @pl.loop(0, ROWS_B // L)
    def _(i):
        zb_v[pl.ds(i * L, L)] = zero16

    @pl.loop(0, GPC)
    def _(g):
        rowids = g * L + iota

        @pl.loop(0, D, unroll=8)
        def _(j):
            plsc.store_scatter(rows0_v, [rowids, jnp.full((L,), j, jnp.int32)],
                               zero16)

    nzc = jnp.where(sid == NS - 1, 8, 7)

    @pl.loop(0, nzc)
    def _(i):
        pltpu.async_copy(rows0_v, h_sh.at[pl.ds(r0 + i * C, C)], semo)

    @pl.when(sid < NS - 1)
    def _():
        pltpu.async_copy(rows0_v.at[pl.ds(0, 64)],
                         h_sh.at[pl.ds(r0 + 560, 64)], semo)

    @pl.when(sid < NS - 1)
    def _():
        pltpu.async_copy(zb_v.at[pl.ds(0, ROWS_A)], pi_sh.at[pl.ds(r0, ROWS_A)],
                         semo)

    @pl.when(sid == NS - 1)
    def _():
        pltpu.async_copy(zb_v, pi_sh.at[pl.ds(r0, ROWS_B)], semo)

    fill(0, 0)
    fill(1, 1)

    @pl.loop(0, nzc)
    def _(i):
        pltpu.make_async_copy(rows0_v, h_sh.at[pl.ds(r0 + i * C, C)], semo).wait()

    @pl.when(sid < NS - 1)
    def _():
        pltpu.make_async_copy(rows0_v.at[pl.ds(0, 64)],
                              h_sh.at[pl.ds(r0 + 560, 64)], semo).wait()
        pltpu.make_async_copy(zb_v.at[pl.ds(0, ROWS_A)],
                              pi_sh.at[pl.ds(r0, ROWS_A)], semo).wait()

    @pl.when(sid == NS - 1)
    def _():
        pltpu.make_async_copy(zb_v, pi_sh.at[pl.ds(r0, ROWS_B)], semo).wait()

    plsc.subcore_barrier()

    wait_idx(0, 0)
    start_gather(0)

    def body(k, b):
        other = 1 - b

        @pl.when(k + 1 < NCHUNK)
        def _():
            wait_idx(k + 1, other)
            start_gather(other)

        off = wid * EPW + k * C
        pltpu.make_async_copy(x_hbm.at[src_v.at[b]], rows_v[b], semr[b]).wait()
        pltpu.make_async_copy(ea_hbm.at[pl.ds(off, C)], ea_v[b], seme[b]).wait()

        @pl.loop(0, GPC)
        def _(g):
            base = g * L
            rowids = base + iota

            def dot_step(j, acc):
                jv = jnp.full((L,), j, jnp.int32)
                xv = plsc.load_gather(rows_v[b], [rowids, jv])
                ev = plsc.load_gather(ea_v[b], [rowids, jv])
                return acc + xv * ev

            dots = pl.loop(0, D, init_carry=jnp.zeros((L,), jnp.float32),
                           unroll=8)(dot_step)
            pe16 = jnp.exp(dots)
            pe_v[b, pl.ds(base, L)] = pe16

            @pl.loop(0, D, unroll=8)
            def _(j):
                jv = jnp.full((L,), j, jnp.int32)
                xv = plsc.load_gather(rows_v[b], [rowids, jv])
                plsc.store_scatter(rows_v[b], [rowids, jv], xv * pe16)

        pltpu.sync_copy(pe_v.at[b], pi_sh.at[dst_v.at[b]], add=True)
        pltpu.sync_copy(rows_v[b], h_sh.at[dst_v.at[b]], add=True)

        @pl.when(k + 2 < NCHUNK)
        def _():
            fill(k + 2, b)

    @pl.loop(0, NCHUNK - 1, step=2)
    def _(k):
        body(k, 0)
        body(k + 1, 1)

    body(NCHUNK - 1, 0)

    plsc.subcore_barrier()

    @pl.loop(0, nzc)
    def _(i):
        bb = i % 2

        @pl.when(bb == 0)
        def _():
            @pl.when(i >= 2)
            def _():
                pltpu.make_async_copy(
                    rows0_v, hp_hbm.at[cid, pl.ds(r0 + (i - 2) * C, C)],
                    semo).wait()

            pltpu.sync_copy(h_sh.at[pl.ds(r0 + i * C, C)], rows0_v)
            pltpu.async_copy(rows0_v, hp_hbm.at[cid, pl.ds(r0 + i * C, C)], semo)

        @pl.when(bb == 1)
        def _():
            @pl.when(i >= 2)
            def _():
                pltpu.make_async_copy(
                    rows1_v, hp_hbm.at[cid, pl.ds(r0 + (i - 2) * C, C)],
                    semo).wait()

            pltpu.sync_copy(h_sh.at[pl.ds(r0 + i * C, C)], rows1_v)
            pltpu.async_copy(rows1_v, hp_hbm.at[cid, pl.ds(r0 + i * C, C)], semo)

    @pl.loop(nzc - 2, nzc)
    def _(i):
        bb = i % 2

        @pl.when(bb == 0)
        def _():
            pltpu.make_async_copy(rows0_v, hp_hbm.at[cid, pl.ds(r0 + i * C, C)],
                                  semo).wait()

        @pl.when(bb == 1)
        def _():
            pltpu.make_async_copy(rows1_v, hp_hbm.at[cid, pl.ds(r0 + i * C, C)],
                                  semo).wait()

    @pl.when(sid < NS - 1)
    def _():
        pltpu.sync_copy(h_sh.at[pl.ds(r0 + 560, 64)], rows0_v.at[pl.ds(0, 64)])
        pltpu.sync_copy(rows0_v.at[pl.ds(0, 64)],
                        hp_hbm.at[cid, pl.ds(r0 + 560, 64)])
        pltpu.sync_copy(pi_sh.at[pl.ds(r0, ROWS_A)], zb_v.at[pl.ds(0, ROWS_A)])
        pltpu.sync_copy(zb_v.at[pl.ds(0, ROWS_A)],
                        pip_hbm.at[pl.ds(cid * N + r0, ROWS_A)])

    @pl.when(sid == NS - 1)
    def _():
        pltpu.sync_copy(pi_sh.at[pl.ds(r0, ROWS_B)], zb_v)
        pltpu.sync_copy(zb_v, pip_hbm.at[pl.ds(cid * N + r0, ROWS_B)])


RPW = 320
RPW_LAST = N - 31 * RPW


@functools.partial(
    pl.kernel,
    out_type=jax.ShapeDtypeStruct((N, D), jnp.float32),
    mesh=_mesh,
    compiler_params=_params,
    scratch_types=[
        pltpu.VMEM((RPW, D), jnp.float32),
        pltpu.VMEM((RPW, D), jnp.float32),
        pltpu.VMEM((RPW,), jnp.float32),
        pltpu.VMEM((RPW,), jnp.float32),
        pltpu.VMEM((RPW,), jnp.float32),
        pltpu.SemaphoreType.DMA,
    ],
)
def _k2(pip_hbm, hp_hbm, hs_hbm, b0_v, b1_v, d0_v, d1_v, rec_v, sem):
    cid = lax.axis_index("c")
    sid = lax.axis_index("s")
    wid = cid * NS + sid
    iota = _iota16()
    base = wid * RPW

    def work(nr):
        pltpu.async_copy(hp_hbm.at[0, pl.ds(base, nr)], b0_v.at[pl.ds(0, nr)], sem)
        pltpu.async_copy(hp_hbm.at[1, pl.ds(base, nr)], b1_v.at[pl.ds(0, nr)], sem)
        pltpu.async_copy(pip_hbm.at[pl.ds(base, nr)], d0_v.at[pl.ds(0, nr)], sem)
        pltpu.async_copy(pip_hbm.at[pl.ds(N + base, nr)], d1_v.at[pl.ds(0, nr)],
                         sem)
        pltpu.make_async_copy(hp_hbm.at[0, pl.ds(base, nr)],
                              b0_v.at[pl.ds(0, nr)], sem).wait()
        pltpu.make_async_copy(hp_hbm.at[1, pl.ds(base, nr)],
                              b1_v.at[pl.ds(0, nr)], sem).wait()
        pltpu.make_async_copy(pip_hbm.at[pl.ds(base, nr)],
                              d0_v.at[pl.ds(0, nr)], sem).wait()
        pltpu.make_async_copy(pip_hbm.at[pl.ds(N + base, nr)],
                              d1_v.at[pl.ds(0, nr)], sem).wait()

        @pl.loop(0, nr // L)
        def _(i):
            s = pl.ds(i * L, L)
            rec_v[s] = 1.0 / (d0_v[s] + d1_v[s])

        @pl.loop(0, nr // L)
        def _(g):
            rowids = g * L + iota
            rec16 = rec_v[pl.ds(g * L, L)]

            @pl.loop(0, D, unroll=8)
            def _(j):
                jv = jnp.full((L,), j, jnp.int32)
                v = (plsc.load_gather(b0_v, [rowids, jv])
                     + plsc.load_gather(b1_v, [rowids, jv]))
                plsc.store_scatter(b0_v, [rowids, jv], v * rec16)

        pltpu.sync_copy(b0_v.at[pl.ds(0, nr)], hs_hbm.at[pl.ds(base, nr)])

    @pl.when(wid < NW - 1)
    def _():
        work(RPW)

    @pl.when(wid == NW - 1)
    def _():
        work(RPW_LAST)


_RB = 1000


def _mm_body(x_r, hs_r, wt_r, b_r, o_r):
    o_r[...] = (
        jnp.dot(x_r[...], wt_r[0:D, :], preferred_element_type=jnp.float32)
        + jnp.dot(hs_r[...], wt_r[D:2 * D, :], preferred_element_type=jnp.float32)
        + b_r[...]
    )


_k3 = pl.pallas_call(
    _mm_body,
    grid=(N // _RB,),
    in_specs=[
        pl.BlockSpec((_RB, D), lambda i: (i, 0)),
        pl.BlockSpec((_RB, D), lambda i: (i, 0)),
        pl.BlockSpec((2 * D, OUT), lambda i: (0, 0)),
        pl.BlockSpec((1, OUT), lambda i: (0, 0)),
    ],
    out_specs=pl.BlockSpec((_RB, OUT), lambda i: (i, 0)),
    out_shape=jax.ShapeDtypeStruct((N, OUT), jnp.float32),
)


@jax.jit
def kernel(x, edge_index, edge_attr, W, b):
    src = edge_index[0]
    dst = edge_index[1]
    pip, hp = _k1(x, src, dst, edge_attr)
    hs = _k2(pip, hp)
    return _k3(x, hs, W.T, b.reshape(1, OUT))


# speedup vs baseline: 6.7744x; 5.4348x over previous
import functools

import jax
import jax.numpy as jnp
from jax import lax
from jax.experimental import pallas as pl
from jax.experimental.pallas import tpu as pltpu
from jax.experimental.pallas import tpu_sc as plsc

N = 10000
E = 320000
D = 128
OUT = 128

NC = 2
NS = 16
L = 16
NW = NC * NS
EPW = E // NW
C = 80
NCHUNK = EPW // C
GPC = C // L

ROWS_A = 624
ROWS_B = N - 15 * ROWS_A

_mesh = plsc.VectorSubcoreMesh(
    core_axis_name="c", subcore_axis_name="s", num_cores=NC, num_subcores=NS
)
_params = pltpu.CompilerParams(needs_layout_passes=False)


def _iota16():
    return lax.broadcasted_iota(jnp.int32, (L,), 0)


@functools.partial(
    pl.kernel,
    out_type=(
        jax.ShapeDtypeStruct((NC * N,), jnp.float32),
        jax.ShapeDtypeStruct((NC, N, D), jnp.float32),
    ),
    mesh=_mesh,
    compiler_params=_params,
    scratch_types=[
        pltpu.VMEM((2, C), jnp.int32),
        pltpu.VMEM((2, C), jnp.int32),
        pltpu.VMEM((C, D), jnp.float32),
        pltpu.VMEM((C, D), jnp.float32),
        pltpu.VMEM((C, D), jnp.float32),
        pltpu.VMEM((C, D), jnp.float32),
        pltpu.VMEM((2, C), jnp.float32),
        pltpu.VMEM((ROWS_B,), jnp.float32),
        pltpu.VMEM_SHARED((N,), jnp.float32),
        pltpu.VMEM_SHARED((N, D), jnp.float32),
        pltpu.SemaphoreType.DMA,
        pltpu.SemaphoreType.DMA,
        pltpu.SemaphoreType.DMA,
        pltpu.SemaphoreType.DMA,
        pltpu.SemaphoreType.DMA,
        pltpu.SemaphoreType.DMA,
        pltpu.SemaphoreType.DMA,
    ],
)
def _k1(x_hbm, src_hbm, dst_hbm, ea_hbm, pip_hbm, hp_hbm,
        src_v, dst_v, rows0_v, rows1_v, ea0_v, ea1_v, pe_v, zb_v,
        pi_sh, h_sh, semi0, semi1, seme0, seme1, semr0, semr1, semo):
    cid = lax.axis_index("c")
    sid = lax.axis_index("s")
    wid = cid * NS + sid
    iota = _iota16()
    zero16 = jnp.zeros((L,), jnp.float32)
    rows_v = (rows0_v, rows1_v)
    ea_v = (ea0_v, ea1_v)
    semi = (semi0, semi1)
    seme = (seme0, seme1)
    semr = (semr0, semr1)

    r0 = sid * ROWS_A

    def fill(k, b):
        off = wid * EPW + k * C
        pltpu.async_copy(src_hbm.at[pl.ds(off, C)], src_v.at[b], semi[b])
        pltpu.async_copy(dst_hbm.at[pl.ds(off, C)], dst_v.at[b], semi[b])
        pltpu.async_copy(ea_hbm.at[pl.ds(off, C)], ea_v[b], seme[b])

    def wait_idx(k, b):
        off = wid * EPW + k * C
        pltpu.make_async_copy(src_hbm.at[pl.ds(off, C)], src_v.at[b], semi[b]).wait()
        pltpu.make_async_copy(dst_hbm.at[pl.ds(off, C)], dst_v.at[b], semi[b]).wait()

    def start_gather(b):
        pltpu.async_copy(x_hbm.at[src_v.at[b]], rows_v[b], semr[b])

    @pl.loop(0, ROWS_B // L)
    def _(i):
        zb_v[pl.ds(i * L, L)] = zero16

    @pl.loop(0, GPC)
    def _(g):
        rowids = g * L + iota

        @pl.loop(0, D, unroll=8)
        def _(j):
            plsc.store_scatter(rows0_v, [rowids, jnp.full((L,), j, jnp.int32)],
                               zero16)

    nzc = jnp.where(sid == NS - 1, 8, 7)

    @pl.loop(0, nzc)
    def _(i):
        pltpu.async_copy(rows0_v, h_sh.at[pl.ds(r0 + i * C, C)], semo)

    @pl.when(sid < NS - 1)
    def _():
        pltpu.async_copy(rows0_v.at[pl.ds(0, 64)],
                         h_sh.at[pl.ds(r0 + 560, 64)], semo)

    @pl.when(sid < NS - 1)
    def _():
        pltpu.async_copy(zb_v.at[pl.ds(0, ROWS_A)], pi_sh.at[pl.ds(r0, ROWS_A)],
                         semo)

    @pl.when(sid == NS - 1)
    def _():
        pltpu.async_copy(zb_v, pi_sh.at[pl.ds(r0, ROWS_B)], semo)

    fill(0, 0)
    fill(1, 1)

    @pl.loop(0, nzc)
    def _(i):
        pltpu.make_async_copy(rows0_v, h_sh.at[pl.ds(r0 + i * C, C)], semo).wait()

    @pl.when(sid < NS - 1)
    def _():
        pltpu.make_async_copy(rows0_v.at[pl.ds(0, 64)],
                              h_sh.at[pl.ds(r0 + 560, 64)], semo).wait()
        pltpu.make_async_copy(zb_v.at[pl.ds(0, ROWS_A)],
                              pi_sh.at[pl.ds(r0, ROWS_A)], semo).wait()

    @pl.when(sid == NS - 1)
    def _():
        pltpu.make_async_copy(zb_v, pi_sh.at[pl.ds(r0, ROWS_B)], semo).wait()

    plsc.subcore_barrier()

    wait_idx(0, 0)
    start_gather(0)

    def body(k, b):
        other = 1 - b

        @pl.when(k + 1 < NCHUNK)
        def _():
            wait_idx(k + 1, other)
            start_gather(other)

        off = wid * EPW + k * C
        pltpu.make_async_copy(x_hbm.at[src_v.at[b]], rows_v[b], semr[b]).wait()
        pltpu.make_async_copy(ea_hbm.at[pl.ds(off, C)], ea_v[b], seme[b]).wait()

        @pl.loop(0, GPC)
        def _(g):
            base = g * L
            dots = jnp.zeros((L,), jnp.float32)
            for e in range(L):
                r = base + e
                ps = [rows_v[b][r, pl.ds(j * L, L)] * ea_v[b][r, pl.ds(j * L, L)]
                      for j in range(D // L)]
                s = ((ps[0] + ps[1]) + (ps[2] + ps[3])) + \
                    ((ps[4] + ps[5]) + (ps[6] + ps[7]))
                for sh in (8, 4, 2, 1):
                    s = s + jnp.take(s, iota ^ sh)
                dots = jnp.where(iota == e, s, dots)
            pe16 = jnp.exp(dots)
            pe_v[b, pl.ds(base, L)] = pe16
            for e in range(L):
                r = base + e
                bvec = jnp.take(pe16, jnp.full((L,), e, jnp.int32))
                for j in range(D // L):
                    s2 = pl.ds(j * L, L)
                    rows_v[b][r, s2] = rows_v[b][r, s2] * bvec

        pltpu.sync_copy(pe_v.at[b], pi_sh.at[dst_v.at[b]], add=True)
        pltpu.sync_copy(rows_v[b], h_sh.at[dst_v.at[b]], add=True)

        @pl.when(k + 2 < NCHUNK)
        def _():
            fill(k + 2, b)

    @pl.loop(0, NCHUNK - 1, step=2)
    def _(k):
        body(k, 0)
        body(k + 1, 1)

    body(NCHUNK - 1, 0)

    plsc.subcore_barrier()

    @pl.loop(0, nzc)
    def _(i):
        bb = i % 2

        @pl.when(bb == 0)
        def _():
            @pl.when(i >= 2)
            def _():
                pltpu.make_async_copy(
                    rows0_v, hp_hbm.at[cid, pl.ds(r0 + (i - 2) * C, C)],
                    semo).wait()

            pltpu.sync_copy(h_sh.at[pl.ds(r0 + i * C, C)], rows0_v)
            pltpu.async_copy(rows0_v, hp_hbm.at[cid, pl.ds(r0 + i * C, C)], semo)

        @pl.when(bb == 1)
        def _():
            @pl.when(i >= 2)
            def _():
                pltpu.make_async_copy(
                    rows1_v, hp_hbm.at[cid, pl.ds(r0 + (i - 2) * C, C)],
                    semo).wait()

            pltpu.sync_copy(h_sh.at[pl.ds(r0 + i * C, C)], rows1_v)
            pltpu.async_copy(rows1_v, hp_hbm.at[cid, pl.ds(r0 + i * C, C)], semo)

    @pl.loop(nzc - 2, nzc)
    def _(i):
        bb = i % 2

        @pl.when(bb == 0)
        def _():
            pltpu.make_async_copy(rows0_v, hp_hbm.at[cid, pl.ds(r0 + i * C, C)],
                                  semo).wait()

        @pl.when(bb == 1)
        def _():
            pltpu.make_async_copy(rows1_v, hp_hbm.at[cid, pl.ds(r0 + i * C, C)],
                                  semo).wait()

    @pl.when(sid < NS - 1)
    def _():
        pltpu.sync_copy(h_sh.at[pl.ds(r0 + 560, 64)], rows0_v.at[pl.ds(0, 64)])
        pltpu.sync_copy(rows0_v.at[pl.ds(0, 64)],
                        hp_hbm.at[cid, pl.ds(r0 + 560, 64)])
        pltpu.sync_copy(pi_sh.at[pl.ds(r0, ROWS_A)], zb_v.at[pl.ds(0, ROWS_A)])
        pltpu.sync_copy(zb_v.at[pl.ds(0, ROWS_A)],
                        pip_hbm.at[pl.ds(cid * N + r0, ROWS_A)])

    @pl.when(sid == NS - 1)
    def _():
        pltpu.sync_copy(pi_sh.at[pl.ds(r0, ROWS_B)], zb_v)
        pltpu.sync_copy(zb_v, pip_hbm.at[pl.ds(cid * N + r0, ROWS_B)])


RPW = 320
RPW_LAST = N - 31 * RPW


@functools.partial(
    pl.kernel,
    out_type=jax.ShapeDtypeStruct((N, D), jnp.float32),
    mesh=_mesh,
    compiler_params=_params,
    scratch_types=[
        pltpu.VMEM((RPW, D), jnp.float32),
        pltpu.VMEM((RPW, D), jnp.float32),
        pltpu.VMEM((RPW,), jnp.float32),
        pltpu.VMEM((RPW,), jnp.float32),
        pltpu.VMEM((RPW,), jnp.float32),
        pltpu.SemaphoreType.DMA,
    ],
)
def _k2(pip_hbm, hp_hbm, hs_hbm, b0_v, b1_v, d0_v, d1_v, rec_v, sem):
    cid = lax.axis_index("c")
    sid = lax.axis_index("s")
    wid = cid * NS + sid
    iota = _iota16()
    base = wid * RPW

    def work(nr):
        pltpu.async_copy(hp_hbm.at[0, pl.ds(base, nr)], b0_v.at[pl.ds(0, nr)], sem)
        pltpu.async_copy(hp_hbm.at[1, pl.ds(base, nr)], b1_v.at[pl.ds(0, nr)], sem)
        pltpu.async_copy(pip_hbm.at[pl.ds(base, nr)], d0_v.at[pl.ds(0, nr)], sem)
        pltpu.async_copy(pip_hbm.at[pl.ds(N + base, nr)], d1_v.at[pl.ds(0, nr)],
                         sem)
        pltpu.make_async_copy(hp_hbm.at[0, pl.ds(base, nr)],
                              b0_v.at[pl.ds(0, nr)], sem).wait()
        pltpu.make_async_copy(hp_hbm.at[1, pl.ds(base, nr)],
                              b1_v.at[pl.ds(0, nr)], sem).wait()
        pltpu.make_async_copy(pip_hbm.at[pl.ds(base, nr)],
                              d0_v.at[pl.ds(0, nr)], sem).wait()
        pltpu.make_async_copy(pip_hbm.at[pl.ds(N + base, nr)],
                              d1_v.at[pl.ds(0, nr)], sem).wait()

        @pl.loop(0, nr // L)
        def _(i):
            s = pl.ds(i * L, L)
            rec_v[s] = 1.0 / (d0_v[s] + d1_v[s])

        @pl.loop(0, nr // L)
        def _(g):
            rowids = g * L + iota
            rec16 = rec_v[pl.ds(g * L, L)]

            @pl.loop(0, D, unroll=8)
            def _(j):
                jv = jnp.full((L,), j, jnp.int32)
                v = (plsc.load_gather(b0_v, [rowids, jv])
                     + plsc.load_gather(b1_v, [rowids, jv]))
                plsc.store_scatter(b0_v, [rowids, jv], v * rec16)

        pltpu.sync_copy(b0_v.at[pl.ds(0, nr)], hs_hbm.at[pl.ds(base, nr)])

    @pl.when(wid < NW - 1)
    def _():
        work(RPW)

    @pl.when(wid == NW - 1)
    def _():
        work(RPW_LAST)


_RB = 1000


def _mm_body(x_r, hs_r, wt_r, b_r, o_r):
    o_r[...] = (
        jnp.dot(x_r[...], wt_r[0:D, :], preferred_element_type=jnp.float32)
        + jnp.dot(hs_r[...], wt_r[D:2 * D, :], preferred_element_type=jnp.float32)
        + b_r[...]
    )


_k3 = pl.pallas_call(
    _mm_body,
    grid=(N // _RB,),
    in_specs=[
        pl.BlockSpec((_RB, D), lambda i: (i, 0)),
        pl.BlockSpec((_RB, D), lambda i: (i, 0)),
        pl.BlockSpec((2 * D, OUT), lambda i: (0, 0)),
        pl.BlockSpec((1, OUT), lambda i: (0, 0)),
    ],
    out_specs=pl.BlockSpec((_RB, OUT), lambda i: (i, 0)),
    out_shape=jax.ShapeDtypeStruct((N, OUT), jnp.float32),
)


@jax.jit
def kernel(x, edge_index, edge_attr, W, b):
    src = edge_index[0]
    dst = edge_index[1]
    pip, hp = _k1(x, src, dst, edge_attr)
    hs = _k2(pip, hp)
    return _k3(x, hs, W.T, b.reshape(1, OUT))


# speedup vs baseline: 8.8940x; 1.3129x over previous
import functools

import jax
import jax.numpy as jnp
from jax import lax
from jax.experimental import pallas as pl
from jax.experimental.pallas import tpu as pltpu
from jax.experimental.pallas import tpu_sc as plsc

N = 10000
E = 320000
D = 128
OUT = 128

NC = 2
NS = 16
L = 16
NW = NC * NS
EPW = E // NW
C = 80
NCHUNK = EPW // C
GPC = C // L

ROWS_A = 624
ROWS_B = N - 15 * ROWS_A

_mesh = plsc.VectorSubcoreMesh(
    core_axis_name="c", subcore_axis_name="s", num_cores=NC, num_subcores=NS
)
_params = pltpu.CompilerParams(needs_layout_passes=False)


def _iota16():
    return lax.broadcasted_iota(jnp.int32, (L,), 0)


@functools.partial(
    pl.kernel,
    out_type=(
        jax.ShapeDtypeStruct((NC * N,), jnp.float32),
        jax.ShapeDtypeStruct((NC, N, D), jnp.float32),
    ),
    mesh=_mesh,
    compiler_params=_params,
    scratch_types=[
        pltpu.VMEM((2, C), jnp.int32),
        pltpu.VMEM((2, C), jnp.int32),
        pltpu.VMEM((2, C), jnp.int32),
        pltpu.VMEM((C, D), jnp.float32),
        pltpu.VMEM((C, D), jnp.float32),
        pltpu.VMEM((C, D), jnp.float32),
        pltpu.VMEM((C, D), jnp.float32),
        pltpu.VMEM((2, C), jnp.float32),
        pltpu.VMEM((ROWS_B,), jnp.float32),
        pltpu.VMEM_SHARED((N,), jnp.float32),
        pltpu.VMEM_SHARED((N, D), jnp.float32),
        pltpu.SemaphoreType.DMA,
        pltpu.SemaphoreType.DMA,
        pltpu.SemaphoreType.DMA,
        pltpu.SemaphoreType.DMA,
        pltpu.SemaphoreType.DMA,
        pltpu.SemaphoreType.DMA,
        pltpu.SemaphoreType.DMA,
        pltpu.SemaphoreType.DMA,
        pltpu.SemaphoreType.DMA,
    ],
)
def _k1(x_hbm, src_hbm, dst_hbm, ea_hbm, pip_hbm, hp_hbm,
        src_v, dst_v, dsts_v, rows0_v, rows1_v, ea0_v, ea1_v, pe_v, zb_v,
        pi_sh, h_sh, semi0, semi1, seme0, seme1, semr0, semr1, semo,
        sems0, sems1):
    cid = lax.axis_index("c")
    sid = lax.axis_index("s")
    wid = cid * NS + sid
    iota = _iota16()
    zero16 = jnp.zeros((L,), jnp.float32)
    rows_v = (rows0_v, rows1_v)
    ea_v = (ea0_v, ea1_v)
    semi = (semi0, semi1)
    seme = (seme0, seme1)
    semr = (semr0, semr1)
    sems = (sems0, sems1)

    def wait_scatters(bb):
        pltpu.make_async_copy(pe_v.at[bb], pi_sh.at[dsts_v.at[bb]],
                              sems[bb]).wait()
        pltpu.make_async_copy(rows_v[bb], h_sh.at[dsts_v.at[bb]],
                              sems[bb]).wait()

    r0 = sid * ROWS_A

    def fill(k, b):
        off = wid * EPW + k * C
        pltpu.async_copy(src_hbm.at[pl.ds(off, C)], src_v.at[b], semi[b])
        pltpu.async_copy(dst_hbm.at[pl.ds(off, C)], dst_v.at[b], semi[b])
        pltpu.async_copy(ea_hbm.at[pl.ds(off, C)], ea_v[b], seme[b])

    def wait_idx(k, b):
        off = wid * EPW + k * C
        pltpu.make_async_copy(src_hbm.at[pl.ds(off, C)], src_v.at[b], semi[b]).wait()
        pltpu.make_async_copy(dst_hbm.at[pl.ds(off, C)], dst_v.at[b], semi[b]).wait()

    def start_gather(b):
        pltpu.async_copy(x_hbm.at[src_v.at[b]], rows_v[b], semr[b])

    @pl.loop(0, ROWS_B // L)
    def _(i):
        zb_v[pl.ds(i * L, L)] = zero16

    @pl.loop(0, GPC)
    def _(g):
        rowids = g * L + iota

        @pl.loop(0, D, unroll=8)
        def _(j):
            plsc.store_scatter(rows0_v, [rowids, jnp.full((L,), j, jnp.int32)],
                               zero16)

    nzc = jnp.where(sid == NS - 1, 8, 7)

    @pl.loop(0, nzc)
    def _(i):
        pltpu.async_copy(rows0_v, h_sh.at[pl.ds(r0 + i * C, C)], semo)

    @pl.when(sid < NS - 1)
    def _():
        pltpu.async_copy(rows0_v.at[pl.ds(0, 64)],
                         h_sh.at[pl.ds(r0 + 560, 64)], semo)

    @pl.when(sid < NS - 1)
    def _():
        pltpu.async_copy(zb_v.at[pl.ds(0, ROWS_A)], pi_sh.at[pl.ds(r0, ROWS_A)],
                         semo)

    @pl.when(sid == NS - 1)
    def _():
        pltpu.async_copy(zb_v, pi_sh.at[pl.ds(r0, ROWS_B)], semo)

    fill(0, 0)
    fill(1, 1)

    @pl.loop(0, nzc)
    def _(i):
        pltpu.make_async_copy(rows0_v, h_sh.at[pl.ds(r0 + i * C, C)], semo).wait()

    @pl.when(sid < NS - 1)
    def _():
        pltpu.make_async_copy(rows0_v.at[pl.ds(0, 64)],
                              h_sh.at[pl.ds(r0 + 560, 64)], semo).wait()
        pltpu.make_async_copy(zb_v.at[pl.ds(0, ROWS_A)],
                              pi_sh.at[pl.ds(r0, ROWS_A)], semo).wait()

    @pl.when(sid == NS - 1)
    def _():
        pltpu.make_async_copy(zb_v, pi_sh.at[pl.ds(r0, ROWS_B)], semo).wait()

    plsc.subcore_barrier()

    wait_idx(0, 0)
    start_gather(0)

    def body(k, b):
        other = 1 - b

        @pl.when(k + 1 < NCHUNK)
        def _():
            @pl.when(k >= 1)
            def _():
                wait_scatters(other)

            wait_idx(k + 1, other)
            start_gather(other)

        off = wid * EPW + k * C
        pltpu.make_async_copy(x_hbm.at[src_v.at[b]], rows_v[b], semr[b]).wait()
        pltpu.make_async_copy(ea_hbm.at[pl.ds(off, C)], ea_v[b], seme[b]).wait()

        @pl.loop(0, GPC)
        def _(g):
            s1 = pl.ds(g * L, L)
            dsts_v[b, s1] = dst_v[b, s1]

        @pl.loop(0, GPC)
        def _(g):
            base = g * L
            dots = jnp.zeros((L,), jnp.float32)
            for e in range(L):
                r = base + e
                ps = [rows_v[b][r, pl.ds(j * L, L)] * ea_v[b][r, pl.ds(j * L, L)]
                      for j in range(D // L)]
                s = ((ps[0] + ps[1]) + (ps[2] + ps[3])) + \
                    ((ps[4] + ps[5]) + (ps[6] + ps[7]))
                for sh in (8, 4, 2, 1):
                    s = s + jnp.take(s, iota ^ sh)
                dots = jnp.where(iota == e, s, dots)
            pe16 = jnp.exp(dots)
            pe_v[b, pl.ds(base, L)] = pe16
            for e in range(L):
                r = base + e
                bvec = jnp.take(pe16, jnp.full((L,), e, jnp.int32))
                for j in range(D // L):
                    s2 = pl.ds(j * L, L)
                    rows_v[b][r, s2] = rows_v[b][r, s2] * bvec

        pltpu.async_copy(pe_v.at[b], pi_sh.at[dsts_v.at[b]], sems[b], add=True)
        pltpu.async_copy(rows_v[b], h_sh.at[dsts_v.at[b]], sems[b], add=True)

        @pl.when(k + 2 < NCHUNK)
        def _():
            fill(k + 2, b)

    @pl.loop(0, NCHUNK - 1, step=2)
    def _(k):
        body(k, 0)
        body(k + 1, 1)

    body(NCHUNK - 1, 0)
    wait_scatters(1)
    wait_scatters(0)

    plsc.subcore_barrier()

    @pl.loop(0, nzc)
    def _(i):
        bb = i % 2

        @pl.when(bb == 0)
        def _():
            @pl.when(i >= 2)
            def _():
                pltpu.make_async_copy(
                    rows0_v, hp_hbm.at[cid, pl.ds(r0 + (i - 2) * C, C)],
                    semo).wait()

            pltpu.sync_copy(h_sh.at[pl.ds(r0 + i * C, C)], rows0_v)
            pltpu.async_copy(rows0_v, hp_hbm.at[cid, pl.ds(r0 + i * C, C)], semo)

        @pl.when(bb == 1)
        def _():
            @pl.when(i >= 2)
            def _():
                pltpu.make_async_copy(
                    rows1_v, hp_hbm.at[cid, pl.ds(r0 + (i - 2) * C, C)],
                    semo).wait()

            pltpu.sync_copy(h_sh.at[pl.ds(r0 + i * C, C)], rows1_v)
            pltpu.async_copy(rows1_v, hp_hbm.at[cid, pl.ds(r0 + i * C, C)], semo)

    @pl.loop(nzc - 2, nzc)
    def _(i):
        bb = i % 2

        @pl.when(bb == 0)
        def _():
            pltpu.make_async_copy(rows0_v, hp_hbm.at[cid, pl.ds(r0 + i * C, C)],
                                  semo).wait()

        @pl.when(bb == 1)
        def _():
            pltpu.make_async_copy(rows1_v, hp_hbm.at[cid, pl.ds(r0 + i * C, C)],
                                  semo).wait()

    @pl.when(sid < NS - 1)
    def _():
        pltpu.sync_copy(h_sh.at[pl.ds(r0 + 560, 64)], rows0_v.at[pl.ds(0, 64)])
        pltpu.sync_copy(rows0_v.at[pl.ds(0, 64)],
                        hp_hbm.at[cid, pl.ds(r0 + 560, 64)])
        pltpu.sync_copy(pi_sh.at[pl.ds(r0, ROWS_A)], zb_v.at[pl.ds(0, ROWS_A)])
        pltpu.sync_copy(zb_v.at[pl.ds(0, ROWS_A)],
                        pip_hbm.at[pl.ds(cid * N + r0, ROWS_A)])

    @pl.when(sid == NS - 1)
    def _():
        pltpu.sync_copy(pi_sh.at[pl.ds(r0, ROWS_B)], zb_v)
        pltpu.sync_copy(zb_v, pip_hbm.at[pl.ds(cid * N + r0, ROWS_B)])


RPW = 320
RPW_LAST = N - 31 * RPW


@functools.partial(
    pl.kernel,
    out_type=jax.ShapeDtypeStruct((N, D), jnp.float32),
    mesh=_mesh,
    compiler_params=_params,
    scratch_types=[
        pltpu.VMEM((RPW, D), jnp.float32),
        pltpu.VMEM((RPW, D), jnp.float32),
        pltpu.VMEM((RPW,), jnp.float32),
        pltpu.VMEM((RPW,), jnp.float32),
        pltpu.VMEM((RPW,), jnp.float32),
        pltpu.SemaphoreType.DMA,
    ],
)
def _k2(pip_hbm, hp_hbm, hs_hbm, b0_v, b1_v, d0_v, d1_v, rec_v, sem):
    cid = lax.axis_index("c")
    sid = lax.axis_index("s")
    wid = cid * NS + sid
    iota = _iota16()
    base = wid * RPW

    def work(nr):
        pltpu.async_copy(hp_hbm.at[0, pl.ds(base, nr)], b0_v.at[pl.ds(0, nr)], sem)
        pltpu.async_copy(hp_hbm.at[1, pl.ds(base, nr)], b1_v.at[pl.ds(0, nr)], sem)
        pltpu.async_copy(pip_hbm.at[pl.ds(base, nr)], d0_v.at[pl.ds(0, nr)], sem)
        pltpu.async_copy(pip_hbm.at[pl.ds(N + base, nr)], d1_v.at[pl.ds(0, nr)],
                         sem)
        pltpu.make_async_copy(hp_hbm.at[0, pl.ds(base, nr)],
                              b0_v.at[pl.ds(0, nr)], sem).wait()
        pltpu.make_async_copy(hp_hbm.at[1, pl.ds(base, nr)],
                              b1_v.at[pl.ds(0, nr)], sem).wait()
        pltpu.make_async_copy(pip_hbm.at[pl.ds(base, nr)],
                              d0_v.at[pl.ds(0, nr)], sem).wait()
        pltpu.make_async_copy(pip_hbm.at[pl.ds(N + base, nr)],
                              d1_v.at[pl.ds(0, nr)], sem).wait()

        @pl.loop(0, nr // L)
        def _(i):
            s = pl.ds(i * L, L)
            rec_v[s] = 1.0 / (d0_v[s] + d1_v[s])

        @pl.loop(0, nr // L)
        def _(g):
            rec16 = rec_v[pl.ds(g * L, L)]
            for e in range(L):
                r = g * L + e
                bvec = jnp.take(rec16, jnp.full((L,), e, jnp.int32))
                for j in range(D // L):
                    s2 = pl.ds(j * L, L)
                    b0_v[r, s2] = (b0_v[r, s2] + b1_v[r, s2]) * bvec

        pltpu.sync_copy(b0_v.at[pl.ds(0, nr)], hs_hbm.at[pl.ds(base, nr)])

    @pl.when(wid < NW - 1)
    def _():
        work(RPW)

    @pl.when(wid == NW - 1)
    def _():
        work(RPW_LAST)


_RB = 1000


def _mm_body(x_r, hs_r, wt_r, b_r, o_r):
    o_r[...] = (
        jnp.dot(x_r[...], wt_r[0:D, :], preferred_element_type=jnp.float32)
        + jnp.dot(hs_r[...], wt_r[D:2 * D, :], preferred_element_type=jnp.float32)
        + b_r[...]
    )


_k3 = pl.pallas_call(
    _mm_body,
    grid=(N // _RB,),
    in_specs=[
        pl.BlockSpec((_RB, D), lambda i: (i, 0)),
        pl.BlockSpec((_RB, D), lambda i: (i, 0)),
        pl.BlockSpec((2 * D, OUT), lambda i: (0, 0)),
        pl.BlockSpec((1, OUT), lambda i: (0, 0)),
    ],
    out_specs=pl.BlockSpec((_RB, OUT), lambda i: (i, 0)),
    out_shape=jax.ShapeDtypeStruct((N, OUT), jnp.float32),
)


@jax.jit
def kernel(x, edge_index, edge_attr, W, b):
    src = edge_index[0]
    dst = edge_index[1]
    pip, hp = _k1(x, src, dst, edge_attr)
    hs = _k2(pip, hp)
    return _k3(x, hs, W.T, b.reshape(1, OUT))


# speedup vs baseline: 12.4874x; 1.4040x over previous
import functools

import jax
import jax.numpy as jnp
from jax import lax
from jax.experimental import pallas as pl
from jax.experimental.pallas import tpu as pltpu
from jax.experimental.pallas import tpu_sc as plsc

N = 10000
E = 320000
D = 128
OUT = 128

NC = 2
NS = 16
L = 16
NW = NC * NS
EPW = E // NW
C = 80
NCHUNK = EPW // C
GPC = C // L

ROWS_A = 624
ROWS_B = N - 15 * ROWS_A

_mesh = plsc.VectorSubcoreMesh(
    core_axis_name="c", subcore_axis_name="s", num_cores=NC, num_subcores=NS
)
_params = pltpu.CompilerParams(needs_layout_passes=False)


def _iota16():
    return lax.broadcasted_iota(jnp.int32, (L,), 0)


@functools.partial(
    pl.kernel,
    out_type=(
        jax.ShapeDtypeStruct((NC * N,), jnp.float32),
        jax.ShapeDtypeStruct((NC, N, D), jnp.float32),
    ),
    mesh=_mesh,
    compiler_params=_params,
    scratch_types=[
        pltpu.VMEM((2, C), jnp.int32),
        pltpu.VMEM((2, C), jnp.int32),
        pltpu.VMEM((2, C), jnp.int32),
        pltpu.VMEM((C, D), jnp.float32),
        pltpu.VMEM((C, D), jnp.float32),
        pltpu.VMEM((C, D), jnp.float32),
        pltpu.VMEM((C, D), jnp.float32),
        pltpu.VMEM((2, C), jnp.float32),
        pltpu.VMEM((ROWS_B,), jnp.float32),
        pltpu.VMEM_SHARED((N,), jnp.float32),
        pltpu.VMEM_SHARED((N, D), jnp.float32),
        pltpu.SemaphoreType.DMA,
        pltpu.SemaphoreType.DMA,
        pltpu.SemaphoreType.DMA,
        pltpu.SemaphoreType.DMA,
        pltpu.SemaphoreType.DMA,
        pltpu.SemaphoreType.DMA,
        pltpu.SemaphoreType.DMA,
        pltpu.SemaphoreType.DMA,
        pltpu.SemaphoreType.DMA,
    ],
)
def _k1(x_hbm, src_hbm, dst_hbm, ea_hbm, pip_hbm, hp_hbm,
        src_v, dst_v, dsts_v, rows0_v, rows1_v, ea0_v, ea1_v, pe_v, zb_v,
        pi_sh, h_sh, semi0, semi1, seme0, seme1, semr0, semr1, semo,
        sems0, sems1):
    cid = lax.axis_index("c")
    sid = lax.axis_index("s")
    wid = cid * NS + sid
    iota = _iota16()
    zero16 = jnp.zeros((L,), jnp.float32)
    rows_v = (rows0_v, rows1_v)
    ea_v = (ea0_v, ea1_v)
    semi = (semi0, semi1)
    seme = (seme0, seme1)
    semr = (semr0, semr1)
    sems = (sems0, sems1)

    def wait_scatters(bb):
        pltpu.make_async_copy(pe_v.at[bb], pi_sh.at[dsts_v.at[bb]],
                              sems[bb]).wait()
        pltpu.make_async_copy(rows_v[bb], h_sh.at[dsts_v.at[bb]],
                              sems[bb]).wait()

    r0 = sid * ROWS_A

    def fill(k, b):
        off = wid * EPW + k * C
        pltpu.async_copy(src_hbm.at[pl.ds(off, C)], src_v.at[b], semi[b])
        pltpu.async_copy(dst_hbm.at[pl.ds(off, C)], dst_v.at[b], semi[b])
        pltpu.async_copy(ea_hbm.at[pl.ds(off, C)], ea_v[b], seme[b])

    def wait_idx(k, b):
        off = wid * EPW + k * C
        pltpu.make_async_copy(src_hbm.at[pl.ds(off, C)], src_v.at[b], semi[b]).wait()
        pltpu.make_async_copy(dst_hbm.at[pl.ds(off, C)], dst_v.at[b], semi[b]).wait()

    def start_gather(b):
        pltpu.async_copy(x_hbm.at[src_v.at[b]], rows_v[b], semr[b])

    @pl.loop(0, ROWS_B // L)
    def _(i):
        zb_v[pl.ds(i * L, L)] = zero16

    @pl.loop(0, GPC)
    def _(g):
        rowids = g * L + iota

        @pl.loop(0, D, unroll=8)
        def _(j):
            plsc.store_scatter(rows0_v, [rowids, jnp.full((L,), j, jnp.int32)],
                               zero16)

    nzc = jnp.where(sid == NS - 1, 8, 7)

    @pl.loop(0, nzc)
    def _(i):
        pltpu.async_copy(rows0_v, h_sh.at[pl.ds(r0 + i * C, C)], semo)

    @pl.when(sid < NS - 1)
    def _():
        pltpu.async_copy(rows0_v.at[pl.ds(0, 64)],
                         h_sh.at[pl.ds(r0 + 560, 64)], semo)

    @pl.when(sid < NS - 1)
    def _():
        pltpu.async_copy(zb_v.at[pl.ds(0, ROWS_A)], pi_sh.at[pl.ds(r0, ROWS_A)],
                         semo)

    @pl.when(sid == NS - 1)
    def _():
        pltpu.async_copy(zb_v, pi_sh.at[pl.ds(r0, ROWS_B)], semo)

    fill(0, 0)
    fill(1, 1)

    @pl.loop(0, nzc)
    def _(i):
        pltpu.make_async_copy(rows0_v, h_sh.at[pl.ds(r0 + i * C, C)], semo).wait()

    @pl.when(sid < NS - 1)
    def _():
        pltpu.make_async_copy(rows0_v.at[pl.ds(0, 64)],
                              h_sh.at[pl.ds(r0 + 560, 64)], semo).wait()
        pltpu.make_async_copy(zb_v.at[pl.ds(0, ROWS_A)],
                              pi_sh.at[pl.ds(r0, ROWS_A)], semo).wait()

    @pl.when(sid == NS - 1)
    def _():
        pltpu.make_async_copy(zb_v, pi_sh.at[pl.ds(r0, ROWS_B)], semo).wait()

    plsc.subcore_barrier()

    wait_idx(0, 0)
    start_gather(0)

    def body(k, b):
        other = 1 - b

        @pl.when(k + 1 < NCHUNK)
        def _():
            @pl.when(k >= 1)
            def _():
                wait_scatters(other)

            wait_idx(k + 1, other)
            start_gather(other)

        off = wid * EPW + k * C
        pltpu.make_async_copy(x_hbm.at[src_v.at[b]], rows_v[b], semr[b]).wait()
        pltpu.make_async_copy(ea_hbm.at[pl.ds(off, C)], ea_v[b], seme[b]).wait()

        @pl.loop(0, GPC)
        def _(g):
            s1 = pl.ds(g * L, L)
            dsts_v[b, s1] = dst_v[b, s1]

        @pl.loop(0, GPC)
        def _(g):
            base = g * L
            pe16 = jnp.zeros((L,), jnp.float32)
            for e in range(L):
                r = base + e
                xs = [rows_v[b][r, pl.ds(j * L, L)] for j in range(D // L)]
                ps = [xs[j] * ea_v[b][r, pl.ds(j * L, L)]
                      for j in range(D // L)]
                s = ((ps[0] + ps[1]) + (ps[2] + ps[3])) + \
                    ((ps[4] + ps[5]) + (ps[6] + ps[7]))
                for sh in (8, 4, 2, 1):
                    s = s + jnp.take(s, iota ^ sh)
                pv = jnp.exp(s)
                for j in range(D // L):
                    rows_v[b][r, pl.ds(j * L, L)] = xs[j] * pv
                pe16 = jnp.where(iota == e, pv, pe16)
            pe_v[b, pl.ds(base, L)] = pe16

        pltpu.async_copy(pe_v.at[b], pi_sh.at[dsts_v.at[b]], sems[b], add=True)
        pltpu.async_copy(rows_v[b], h_sh.at[dsts_v.at[b]], sems[b], add=True)

        @pl.when(k + 2 < NCHUNK)
        def _():
            fill(k + 2, b)

    @pl.loop(0, NCHUNK - 1, step=2)
    def _(k):
        body(k, 0)
        body(k + 1, 1)

    body(NCHUNK - 1, 0)
    wait_scatters(1)
    wait_scatters(0)

    plsc.subcore_barrier()

    @pl.loop(0, nzc)
    def _(i):
        bb = i % 2

        @pl.when(bb == 0)
        def _():
            @pl.when(i >= 2)
            def _():
                pltpu.make_async_copy(
                    rows0_v, hp_hbm.at[cid, pl.ds(r0 + (i - 2) * C, C)],
                    semo).wait()

            pltpu.sync_copy(h_sh.at[pl.ds(r0 + i * C, C)], rows0_v)
            pltpu.async_copy(rows0_v, hp_hbm.at[cid, pl.ds(r0 + i * C, C)], semo)

        @pl.when(bb == 1)
        def _():
            @pl.when(i >= 2)
            def _():
                pltpu.make_async_copy(
                    rows1_v, hp_hbm.at[cid, pl.ds(r0 + (i - 2) * C, C)],
                    semo).wait()

            pltpu.sync_copy(h_sh.at[pl.ds(r0 + i * C, C)], rows1_v)
            pltpu.async_copy(rows1_v, hp_hbm.at[cid, pl.ds(r0 + i * C, C)], semo)

    @pl.loop(nzc - 2, nzc)
    def _(i):
        bb = i % 2

        @pl.when(bb == 0)
        def _():
            pltpu.make_async_copy(rows0_v, hp_hbm.at[cid, pl.ds(r0 + i * C, C)],
                                  semo).wait()

        @pl.when(bb == 1)
        def _():
            pltpu.make_async_copy(rows1_v, hp_hbm.at[cid, pl.ds(r0 + i * C, C)],
                                  semo).wait()

    @pl.when(sid < NS - 1)
    def _():
        pltpu.sync_copy(h_sh.at[pl.ds(r0 + 560, 64)], rows0_v.at[pl.ds(0, 64)])
        pltpu.sync_copy(rows0_v.at[pl.ds(0, 64)],
                        hp_hbm.at[cid, pl.ds(r0 + 560, 64)])
        pltpu.sync_copy(pi_sh.at[pl.ds(r0, ROWS_A)], zb_v.at[pl.ds(0, ROWS_A)])
        pltpu.sync_copy(zb_v.at[pl.ds(0, ROWS_A)],
                        pip_hbm.at[pl.ds(cid * N + r0, ROWS_A)])

    @pl.when(sid == NS - 1)
    def _():
        pltpu.sync_copy(pi_sh.at[pl.ds(r0, ROWS_B)], zb_v)
        pltpu.sync_copy(zb_v, pip_hbm.at[pl.ds(cid * N + r0, ROWS_B)])


_RB = 1000


def _mm_body(x_r, h0_r, h1_r, p0_r, p1_r, wt_r, b_r, o_r):
    h = (h0_r[...] + h1_r[...]) * (1.0 / (p0_r[...] + p1_r[...]))
    o_r[...] = (
        jnp.dot(x_r[...], wt_r[0:D, :], preferred_element_type=jnp.float32)
        + jnp.dot(h, wt_r[D:2 * D, :], preferred_element_type=jnp.float32)
        + b_r[...]
    )


_k3 = pl.pallas_call(
    _mm_body,
    grid=(N // _RB,),
    in_specs=[
        pl.BlockSpec((_RB, D), lambda i: (i, 0)),
        pl.BlockSpec((_RB, D), lambda i: (i, 0)),
        pl.BlockSpec((_RB, D), lambda i: (i, 0)),
        pl.BlockSpec((_RB, 1), lambda i: (i, 0)),
        pl.BlockSpec((_RB, 1), lambda i: (i, 0)),
        pl.BlockSpec((2 * D, OUT), lambda i: (0, 0)),
        pl.BlockSpec((1, OUT), lambda i: (0, 0)),
    ],
    out_specs=pl.BlockSpec((_RB, OUT), lambda i: (i, 0)),
    out_shape=jax.ShapeDtypeStruct((N, OUT), jnp.float32),
)


@jax.jit
def kernel(x, edge_index, edge_attr, W, b):
    src = edge_index[0]
    dst = edge_index[1]
    pip, hp = _k1(x, src, dst, edge_attr)
    p0 = pip[:N].reshape(N, 1)
    p1 = pip[N:].reshape(N, 1)
    return _k3(x, hp[0], hp[1], p0, p1, W.T, b.reshape(1, OUT))
